# Initial kernel scaffold; baseline (speedup 1.0000x reference)
import functools

import numpy as np
import jax
import jax.numpy as jnp
from jax import lax
from jax.experimental import pallas as pl
from jax.experimental.pallas import tpu as pltpu
from jax.experimental.pallas import tpu_sc as plsc

_NUM_LEVELS = 16
_F = 2
_LOG2_T = 19
_T = 2 ** _LOG2_T
_MASK = _T - 1
_N = 524288
_BASE_RES = 16
_SCALE = float(np.exp2(np.log2(2048 / 16) / (_NUM_LEVELS - 1)))
_RES = [int(np.floor(_BASE_RES * (_SCALE ** l))) for l in range(_NUM_LEVELS)]
_PRIMES_I32 = [1, 2654435761 - (1 << 32), 805459861]

_NC, _NS = 2, 16
_NW = _NC * _NS


def _build(n_points, c, interpret=False):
    pw = n_points // _NW
    nchunk = pw // c
    ngrp = c // 16
    assert pw % c == 0 and c % 16 == 0

    mesh = plsc.VectorSubcoreMesh(core_axis_name="c", subcore_axis_name="s")

    @functools.partial(
        pl.kernel,
        out_type=jax.ShapeDtypeStruct((n_points, _NUM_LEVELS * _F), jnp.float32),
        mesh=mesh,
        interpret=interpret,
        scratch_types=[
            pltpu.VMEM((c,), jnp.float32),
            pltpu.VMEM((c,), jnp.float32),
            pltpu.VMEM((c,), jnp.float32),
            pltpu.VMEM((8 * c,), jnp.int32),
            pltpu.VMEM((8 * c,), jnp.int32),
            pltpu.VMEM((8 * c, _F), jnp.float32),
            pltpu.VMEM((8 * c, _F), jnp.float32),
            pltpu.VMEM((8 * c,), jnp.float32),
            pltpu.VMEM((8 * c,), jnp.float32),
            pltpu.VMEM((c, _NUM_LEVELS * _F), jnp.float32),
            pltpu.SemaphoreType.DMA,
            pltpu.SemaphoreType.DMA,
        ],
    )
    def hashgrid(xt, tab, out, xvx, xvy, xvz, idx0, idx1, rows0, rows1,
                 w0, w1, outv, sem0, sem1):
        wid = lax.axis_index("s") * _NC + lax.axis_index("c")
        idxb = (idx0, idx1)
        rowsb = (rows0, rows1)
        wb = (w0, w1)
        sems = (sem0, sem1)
        ii = lax.iota(jnp.int32, 16)

        def pass1(l, b):
            res_f = jnp.float32(_RES[l])
            lofs = jnp.int32(l * _T)

            @pl.loop(0, ngrp)
            def _(g):
                off = pl.multiple_of(g * 16, 16)
                cpair = []
                wpair = []
                for j, xv in enumerate((xvx, xvy, xvz)):
                    p = xv[pl.ds(off, 16)] * res_f
                    pi = p.astype(jnp.int32)
                    fr = p - pi.astype(jnp.float32)
                    prime = _PRIMES_I32[j]
                    c0 = pi if prime == 1 else pi * jnp.int32(prime)
                    c1 = c0 + jnp.int32(prime)
                    cpair.append((c0, c1))
                    wpair.append((jnp.float32(1.0) - fr, fr))
                exy = [[cpair[0][a] ^ cpair[1][d] for d in range(2)]
                       for a in range(2)]
                wxy = [[wpair[0][a] * wpair[1][d] for d in range(2)]
                       for a in range(2)]
                for cor in range(8):
                    dx, dy, dz = (cor >> 2) & 1, (cor >> 1) & 1, cor & 1
                    h = ((exy[dx][dy] ^ cpair[2][dz]) & jnp.int32(_MASK)) + lofs
                    idxb[b][pl.ds(cor * c + off, 16)] = h
                    wb[b][pl.ds(cor * c + off, 16)] = wxy[dx][dy] * wpair[2][dz]

        def accum(l, b):
            col0 = jnp.zeros((16,), jnp.int32)
            col1 = col0 + 1

            @pl.loop(0, ngrp)
            def _(g):
                off = pl.multiple_of(g * 16, 16)
                pvec = off + ii
                acc0 = jnp.zeros((16,), jnp.float32)
                acc1 = jnp.zeros((16,), jnp.float32)
                for cor in range(8):
                    wv = wb[b][pl.ds(cor * c + off, 16)]
                    rvec = cor * c + pvec
                    f0 = plsc.load_gather(rowsb[b], [rvec, col0])
                    f1 = plsc.load_gather(rowsb[b], [rvec, col1])
                    acc0 = acc0 + wv * f0
                    acc1 = acc1 + wv * f1
                plsc.store_scatter(
                    outv, [pvec, jnp.full((16,), 2 * l, jnp.int32)], acc0)
                plsc.store_scatter(
                    outv, [pvec, jnp.full((16,), 2 * l + 1, jnp.int32)], acc1)

        @pl.loop(0, nchunk)
        def _(ch):
            base = wid * pw + ch * c
            pltpu.sync_copy(xt.at[0, pl.ds(base, c)], xvx)
            pltpu.sync_copy(xt.at[1, pl.ds(base, c)], xvy)
            pltpu.sync_copy(xt.at[2, pl.ds(base, c)], xvz)
            cops = [None, None]
            for l in range(_NUM_LEVELS):
                b = l & 1
                pass1(l, b)
                cops[b] = pltpu.async_copy(tab.at[idxb[b]], rowsb[b], sems[b])
                if l > 0:
                    cops[1 - b].wait()
                    accum(l - 1, 1 - b)
            cops[(_NUM_LEVELS - 1) & 1].wait()
            accum(_NUM_LEVELS - 1, (_NUM_LEVELS - 1) & 1)
            pltpu.sync_copy(outv, out.at[pl.ds(base, c), :])

    return hashgrid


_CHUNK = 1024
_hashgrid_sc = _build(_N, _CHUNK)


def kernel(x, table, bound):
    xn = (x + bound) / (2 * bound)
    xt = xn.T
    tab = table.reshape(_NUM_LEVELS * _T, _F)
    return _hashgrid_sc(xt, tab)


# speedup vs baseline: 35.2868x; 35.2868x over previous
import functools

import numpy as np
import jax
import jax.numpy as jnp
from jax import lax
from jax.experimental import pallas as pl
from jax.experimental.pallas import tpu as pltpu
from jax.experimental.pallas import tpu_sc as plsc

_NUM_LEVELS = 16
_F = 2
_LOG2_T = 19
_T = 2 ** _LOG2_T
_MASK = _T - 1
_N = 524288
_BASE_RES = 16
_SCALE = float(np.exp2(np.log2(2048 / 16) / (_NUM_LEVELS - 1)))
_RES = [int(np.floor(_BASE_RES * (_SCALE ** l))) for l in range(_NUM_LEVELS)]
_PRIMES_I32 = [1, 2654435761 - (1 << 32), 805459861]

_NC, _NS = 2, 16
_NW = _NC * _NS
_OUTD = _NUM_LEVELS * _F


def _build(n_points, c, interpret=False):
    pw = n_points // _NW
    nchunk = pw // c
    ngrp = c // 16
    assert pw % c == 0 and c % 16 == 0

    mesh = plsc.VectorSubcoreMesh(core_axis_name="c", subcore_axis_name="s",
                                  num_cores=_NC, num_subcores=_NS)

    @functools.partial(
        pl.kernel,
        out_type=jax.ShapeDtypeStruct((_OUTD * n_points,), jnp.float32),
        mesh=mesh,
        interpret=interpret,
        scratch_types=[
            pltpu.VMEM((c,), jnp.float32),
            pltpu.VMEM((c,), jnp.float32),
            pltpu.VMEM((c,), jnp.float32),
            pltpu.VMEM((16 * c,), jnp.int32),
            pltpu.VMEM((16 * c,), jnp.int32),
            pltpu.VMEM((16 * c,), jnp.float32),
            pltpu.VMEM((16 * c,), jnp.float32),
            pltpu.VMEM((8 * c,), jnp.float32),
            pltpu.VMEM((8 * c,), jnp.float32),
            pltpu.VMEM((_OUTD * c,), jnp.float32),
            pltpu.SemaphoreType.DMA,
            pltpu.SemaphoreType.DMA,
        ],
    )
    def hashgrid(xtx, xty, xtz, tab, out, xvx, xvy, xvz, idx0, idx1,
                 rows0, rows1, w0, w1, outv, sem0, sem1):
        wid = lax.axis_index("s") * _NC + lax.axis_index("c")
        idxb = (idx0, idx1)
        rowsb = (rows0, rows1)
        wb = (w0, w1)
        sems = (sem0, sem1)

        def pass1(l, b):
            res_f = jnp.float32(_RES[l])
            lofs2 = jnp.int32(2 * l * _T)

            @pl.loop(0, ngrp)
            def _(g):
                off = pl.multiple_of(g * 16, 16)
                cpair = []
                wpair = []
                for j, xv in enumerate((xvx, xvy, xvz)):
                    p = xv[pl.ds(off, 16)] * res_f
                    pi = p.astype(jnp.int32)
                    fr = p - pi.astype(jnp.float32)
                    prime = _PRIMES_I32[j]
                    c0 = pi if prime == 1 else pi * jnp.int32(prime)
                    c1 = c0 + jnp.int32(prime)
                    cpair.append((c0, c1))
                    wpair.append((jnp.float32(1.0) - fr, fr))
                exy = [[cpair[0][a] ^ cpair[1][d] for d in range(2)]
                       for a in range(2)]
                wxy = [[wpair[0][a] * wpair[1][d] for d in range(2)]
                       for a in range(2)]
                for cor in range(8):
                    dx, dy, dz = (cor >> 2) & 1, (cor >> 1) & 1, cor & 1
                    h = (exy[dx][dy] ^ cpair[2][dz]) & jnp.int32(_MASK)
                    e0 = h * 2 + lofs2
                    idxb[b][pl.ds(cor * c + off, 16)] = e0
                    idxb[b][pl.ds((8 + cor) * c + off, 16)] = e0 + 1
                    wb[b][pl.ds(cor * c + off, 16)] = wxy[dx][dy] * wpair[2][dz]

        def accum(l, b):
            @pl.loop(0, ngrp)
            def _(g):
                off = pl.multiple_of(g * 16, 16)
                acc0 = jnp.zeros((16,), jnp.float32)
                acc1 = jnp.zeros((16,), jnp.float32)
                for cor in range(8):
                    wv = wb[b][pl.ds(cor * c + off, 16)]
                    f0 = rowsb[b][pl.ds(cor * c + off, 16)]
                    f1 = rowsb[b][pl.ds((8 + cor) * c + off, 16)]
                    acc0 = acc0 + wv * f0
                    acc1 = acc1 + wv * f1
                outv[pl.ds((2 * l) * c + off, 16)] = acc0
                outv[pl.ds((2 * l + 1) * c + off, 16)] = acc1

        @pl.loop(0, nchunk)
        def _(ch):
            base = wid * pw + ch * c
            pltpu.sync_copy(xtx.at[pl.ds(base, c)], xvx)
            pltpu.sync_copy(xty.at[pl.ds(base, c)], xvy)
            pltpu.sync_copy(xtz.at[pl.ds(base, c)], xvz)
            cops = [None, None]
            for l in range(_NUM_LEVELS):
                b = l & 1
                pass1(l, b)
                cops[b] = pltpu.async_copy(tab.at[idxb[b]], rowsb[b], sems[b])
                if l > 0:
                    cops[1 - b].wait()
                    accum(l - 1, 1 - b)
            cops[(_NUM_LEVELS - 1) & 1].wait()
            accum(_NUM_LEVELS - 1, (_NUM_LEVELS - 1) & 1)
            for r in range(_OUTD):
                pltpu.sync_copy(outv.at[pl.ds(r * c, c)],
                                out.at[pl.ds(r * n_points + base, c)])

    return hashgrid


_CHUNK = 1024


@functools.lru_cache(maxsize=None)
def _get_hashgrid():
    return _build(_N, _CHUNK)


def kernel(x, table, bound):
    xn = (x + bound) / (2 * bound)
    tab = table.reshape(_NUM_LEVELS * _T * _F)
    o = _get_hashgrid()(xn[:, 0], xn[:, 1], xn[:, 2], tab)
    return o.reshape(_OUTD, _N).T
